# baseline (speedup 1.0000x reference)
import functools

import jax
import jax.numpy as jnp
from jax import lax
from jax.experimental import pallas as pl
from jax.experimental.pallas import tpu as pltpu
from jax.experimental.pallas import tpu_sc as plsc

NC = 2
NS = 16
NW = NC * NS

EMB_DIM = 64
KB = 4
NBUF = 4


def _make_gather(batch: int, hist: int):
  b_per_w = batch // NW
  n_super = b_per_w // KB
  n_groups = n_super // NBUF
  mesh = plsc.VectorSubcoreMesh(core_axis_name="c", subcore_axis_name="s")

  scratch = [pltpu.VMEM((b_per_w, hist), jnp.int32)]
  scratch += [pltpu.VMEM((KB, hist, EMB_DIM), jnp.float32) for _ in range(NBUF)]
  scratch += [pltpu.SemaphoreType.DMA for _ in range(2 * NBUF)]

  @functools.partial(
      pl.kernel,
      out_type=jax.ShapeDtypeStruct((batch, hist, EMB_DIM), jnp.float32),
      mesh=mesh,
      scratch_types=scratch,
      compiler_params=pltpu.CompilerParams(use_tc_tiling_on_sc=False),
  )
  def gather_kernel(table_hbm, idx_hbm, out_hbm, idx_v, *rest):
    bufs = rest[:NBUF]
    gsems = rest[NBUF:2 * NBUF]
    wsems = rest[2 * NBUF:]
    wid = lax.axis_index("s") * NC + lax.axis_index("c")
    base = wid * b_per_w
    pltpu.sync_copy(idx_hbm.at[pl.ds(base, b_per_w)], idx_v)

    def issue_gathers(s, b):
      for k in range(KB):
        pltpu.async_copy(
            table_hbm.at[idx_v.at[s * KB + k]],
            bufs[b].at[k],
            gsems[b])

    for b in range(NBUF):
      issue_gathers(jnp.int32(b), b)

    def group(gi, carry):
      s0 = gi * NBUF
      for b in range(NBUF):
        s = s0 + b
        for k in range(KB):
          pltpu.make_async_copy(table_hbm.at[idx_v.at[s * KB + k]],
                                bufs[b].at[k], gsems[b]).wait()
        wr = pltpu.make_async_copy(
            bufs[b], out_hbm.at[pl.ds(base + s * KB, KB)], wsems[b])
        wr.start()
        wr.wait()

        @pl.when(gi < n_groups - 1)
        def _():
          issue_gathers(s + NBUF, b)

      return carry

    lax.fori_loop(0, n_groups, group, 0)

  return gather_kernel


def kernel(token_ids, embedding_matrix):
  batch, hist = token_ids.shape
  idx = token_ids.astype(jnp.int32)
  return _make_gather(batch, hist)(embedding_matrix, idx)


# speedup vs baseline: 1.0006x; 1.0006x over previous
import functools

import jax
import jax.numpy as jnp
from jax import lax
from jax.experimental import pallas as pl
from jax.experimental.pallas import tpu as pltpu
from jax.experimental.pallas import tpu_sc as plsc

NC = 2
NS = 16
NW = NC * NS

EMB_DIM = 64
KB = 2
NBUF = 8


def _make_gather(batch: int, hist: int):
  b_per_w = batch // NW
  n_super = b_per_w // KB
  n_groups = n_super // NBUF
  mesh = plsc.VectorSubcoreMesh(core_axis_name="c", subcore_axis_name="s")

  scratch = [pltpu.VMEM((b_per_w, hist), jnp.int32)]
  scratch += [pltpu.VMEM((KB, hist, EMB_DIM), jnp.float32) for _ in range(NBUF)]
  scratch += [pltpu.SemaphoreType.DMA for _ in range(2 * NBUF)]

  @functools.partial(
      pl.kernel,
      out_type=jax.ShapeDtypeStruct((batch, hist, EMB_DIM), jnp.float32),
      mesh=mesh,
      scratch_types=scratch,
      compiler_params=pltpu.CompilerParams(use_tc_tiling_on_sc=False),
  )
  def gather_kernel(table_hbm, idx_hbm, out_hbm, idx_v, *rest):
    bufs = rest[:NBUF]
    gsems = rest[NBUF:2 * NBUF]
    wsems = rest[2 * NBUF:]
    wid = lax.axis_index("s") * NC + lax.axis_index("c")
    base = wid * b_per_w
    pltpu.sync_copy(idx_hbm.at[pl.ds(base, b_per_w)], idx_v)

    def issue_gathers(s, b):
      for k in range(KB):
        pltpu.async_copy(
            table_hbm.at[idx_v.at[s * KB + k]],
            bufs[b].at[k],
            gsems[b])

    for b in range(NBUF):
      issue_gathers(jnp.int32(b), b)

    def group(gi, carry):
      s0 = gi * NBUF
      for b in range(NBUF):
        s = s0 + b
        for k in range(KB):
          pltpu.make_async_copy(table_hbm.at[idx_v.at[s * KB + k]],
                                bufs[b].at[k], gsems[b]).wait()
        wr = pltpu.make_async_copy(
            bufs[b], out_hbm.at[pl.ds(base + s * KB, KB)], wsems[b])
        wr.start()
        wr.wait()

        @pl.when(gi < n_groups - 1)
        def _():
          issue_gathers(s + NBUF, b)

      return carry

    lax.fori_loop(0, n_groups, group, 0)

  return gather_kernel


def kernel(token_ids, embedding_matrix):
  batch, hist = token_ids.shape
  idx = token_ids.astype(jnp.int32)
  return _make_gather(batch, hist)(embedding_matrix, idx)
